# Initial kernel scaffold; baseline (speedup 1.0000x reference)
import jax
import jax.numpy as jnp
from jax.experimental import pallas as pl


def kernel(x, te, pe):
    raise NotImplementedError("write your pallas kernel here")


# speedup vs baseline: 1.1629x; 1.1629x over previous
import functools

import jax
import jax.numpy as jnp
from jax import lax
from jax.experimental import pallas as pl
from jax.experimental.pallas import tpu as pltpu
from jax.experimental.pallas import tpu_sc as plsc

D = 768
B = 4
T = 2048

NC = 2
NS = 16
L = 16
NW = NC * NS
TPW = T // NW
CH = 32
CPB = TPW // CH
NCHUNK = B * CPB
NBUF = 3


def _embed_body(x_hbm, te_hbm, pe_hbm, out_hbm,
                idx_v, pe_v, buf0, buf1, buf2,
                psem, gsem0, gsem1, gsem2, ssem0, ssem1, ssem2):
    cid = lax.axis_index("c")
    sid = lax.axis_index("s")
    wid = sid * NC + cid
    t0 = wid * TPW

    bufs = (buf0, buf1, buf2)
    gsems = (gsem0, gsem1, gsem2)
    ssems = (ssem0, ssem1, ssem2)

    pe_cp = pltpu.async_copy(pe_hbm.at[pl.ds(t0, TPW)], pe_v, psem)

    for b in range(B):
        pltpu.sync_copy(x_hbm.at[pl.ds(b * T + t0, TPW)],
                        idx_v.at[pl.ds(b * TPW, TPW)])

    def start_gather(j):
        return pltpu.async_copy(
            te_hbm.at[idx_v.at[pl.ds(j * CH, CH)]],
            bufs[j % NBUF], gsems[j % NBUF])

    gathers = {}
    stores = {}
    gathers[0] = start_gather(0)
    gathers[1] = start_gather(1)
    gathers[2] = start_gather(2)
    pe_cp.wait()

    for j in range(NCHUNK):
        buf = bufs[j % NBUF]
        gathers[j].wait()

        toff = (j % CPB) * CH

        @pl.loop(0, CH)
        def _(r):
            for c in range(0, D, L):
                plsc.addupdate(buf.at[r, pl.ds(c, L)],
                               pe_v[toff + r, pl.ds(c, L)])

        b = j // CPB
        dst = b * T + t0 + toff
        stores[j] = pltpu.async_copy(buf, out_hbm.at[pl.ds(dst, CH)],
                                     ssems[j % NBUF])

        nxt = j + NBUF - 1
        if nxt >= NBUF and nxt < NCHUNK:
            stores[nxt - NBUF].wait()
            gathers[nxt] = start_gather(nxt)

    for j in range(NCHUNK - NBUF, NCHUNK):
        if j >= 0:
            stores[j].wait()


@jax.jit
def _embed(x_flat, te, pe):
    mesh = plsc.VectorSubcoreMesh(core_axis_name="c", subcore_axis_name="s")
    run = pl.kernel(
        _embed_body,
        out_type=jax.ShapeDtypeStruct((B * T, D), jnp.float32),
        mesh=mesh,
        scratch_types=[
            pltpu.VMEM((B * TPW,), jnp.int32),
            pltpu.VMEM((TPW, D), jnp.float32),
            pltpu.VMEM((CH, D), jnp.float32),
            pltpu.VMEM((CH, D), jnp.float32),
            pltpu.VMEM((CH, D), jnp.float32),
            pltpu.SemaphoreType.DMA,
            pltpu.SemaphoreType.DMA,
            pltpu.SemaphoreType.DMA,
            pltpu.SemaphoreType.DMA,
            pltpu.SemaphoreType.DMA,
            pltpu.SemaphoreType.DMA,
            pltpu.SemaphoreType.DMA,
        ],
    )
    return run(x_flat, te, pe)


def kernel(x, te, pe):
    x_flat = x.reshape(B * T).astype(jnp.int32)
    out = _embed(x_flat, te.astype(jnp.float32), pe.astype(jnp.float32))
    return out.reshape(B, T, D)
